# baseline (speedup 1.0000x reference)
import jax
import jax.numpy as jnp
from jax.experimental import pallas as pl
from jax.experimental.pallas import tpu as pltpu

_RBLK = 1024


def _add_kernel(x_ref, pos_ref, o_ref):
    i = pl.program_id(0)
    seq_blocks = pos_ref.shape[0] // _RBLK
    j = jax.lax.rem(i, seq_blocks)
    o_ref[...] = x_ref[...] + pos_ref[pl.ds(j * _RBLK, _RBLK), :]


def kernel(x, pos_table):
    batch, seq_len, d_model = x.shape
    xf = x.reshape(batch * seq_len, d_model)
    grid = (batch * seq_len // _RBLK,)
    out = pl.pallas_call(
        _add_kernel,
        grid=grid,
        in_specs=[
            pl.BlockSpec((_RBLK, d_model), lambda i: (i, 0)),
            pl.BlockSpec((pos_table.shape[0], d_model), lambda i: (0, 0)),
        ],
        out_specs=pl.BlockSpec((_RBLK, d_model), lambda i: (i, 0)),
        out_shape=jax.ShapeDtypeStruct((batch * seq_len, d_model), x.dtype),
        compiler_params=pltpu.CompilerParams(
            dimension_semantics=("arbitrary",),
        ),
    )(xf, pos_table)
    return out.reshape(batch, seq_len, d_model)
